# baseline (speedup 1.0000x reference)
import functools

import jax
import jax.numpy as jnp
from jax import lax
from jax.experimental import pallas as pl
from jax.experimental.pallas import tpu as pltpu
from jax.experimental.pallas import tpu_sc as plsc

NUM_CHARS = 1000
EMB_DIM = 64
EMB_PAD = 128
BATCH = 4096
SEQ = 20
SEQ_PAD = 24
N_EROWS = BATCH * SEQ_PAD


_INFO = plsc.get_sparse_core_info()
_NC = _INFO.num_cores
_NS = _INFO.num_subcores
_NW = _NC * _NS
_RW = N_EROWS // _NW
_CHUNK = 384
_NCHUNK = _RW // _CHUNK
_NB = 2
_NITER = _NCHUNK // _NB


def _gather_body(emb_hbm, idx_hbm, e_hbm, idx_v, *scratch):
    bufs = scratch[:_NB]
    gsem = scratch[_NB:2 * _NB]
    ssem = scratch[2 * _NB:3 * _NB]
    wid = lax.axis_index("s") * _NC + lax.axis_index("c")
    base = wid * _RW
    pltpu.sync_copy(idx_hbm.at[pl.ds(base, _RW)], idx_v)

    def g_copy(b, c):
        off = pl.multiple_of(c * _CHUNK, 8)
        return pltpu.make_async_copy(
            emb_hbm.at[idx_v.at[pl.ds(off, _CHUNK)]], bufs[b], gsem[b])

    def s_copy(b, c):
        off = pl.multiple_of(c * _CHUNK, 8)
        return pltpu.make_async_copy(
            bufs[b], e_hbm.at[pl.ds(base + off, _CHUNK)], ssem[b])

    for b in range(_NB):
        g_copy(b, b).start()

    def step(i, fire_next):
        for b in range(_NB):
            c = i * _NB + b
            g_copy(b, c).wait()
            s_copy(b, c).start()
            s_copy(b, c).wait()
            if fire_next:
                g_copy(b, c + _NB).start()
        return 0

    lax.fori_loop(0, _NITER - 1, lambda i, _: step(i, True), 0)
    step(_NITER - 1, False)


def _gather_e(emb_pad, idx_flat):
    mesh = plsc.VectorSubcoreMesh(core_axis_name="c", subcore_axis_name="s")
    return pl.kernel(
        _gather_body,
        out_type=jax.ShapeDtypeStruct((N_EROWS, EMB_PAD), jnp.float32),
        mesh=mesh,
        scratch_types=[
            pltpu.VMEM((_RW,), jnp.int32),
        ] + [pltpu.VMEM((_CHUNK, EMB_PAD), jnp.float32)] * _NB
          + [pltpu.SemaphoreType.DMA] * (2 * _NB),
    )(emb_pad, idx_flat)


_BB = 16
_GRID = BATCH // _BB


def _mm_body(e_ref, w_ref, b_ref, out_ref):
    res = lax.dot_general(
        e_ref[...], w_ref[...],
        dimension_numbers=(((1,), (1,)), ((), ())),
        preferred_element_type=jnp.float32,
    ) + b_ref[...]
    for bb in range(_BB):
        out_ref[bb] = res[bb * SEQ_PAD:bb * SEQ_PAD + SEQ, :]


def _linear(e, W_pad, b):
    return pl.pallas_call(
        _mm_body,
        grid=(_GRID,),
        in_specs=[
            pl.BlockSpec((_BB * SEQ_PAD, EMB_PAD), lambda i: (i, 0)),
            pl.BlockSpec((NUM_CHARS, EMB_PAD), lambda i: (0, 0)),
            pl.BlockSpec((1, NUM_CHARS), lambda i: (0, 0)),
        ],
        out_specs=pl.BlockSpec((_BB, SEQ, NUM_CHARS), lambda i: (i, 0, 0)),
        out_shape=jax.ShapeDtypeStruct((BATCH, SEQ, NUM_CHARS), jnp.float32),
    )(e, W_pad, b.reshape(1, NUM_CHARS))


def kernel(x, emb_table, W, b):
    emb_pad = jnp.pad(emb_table, ((0, 0), (0, EMB_PAD - EMB_DIM)))
    w_pad = jnp.pad(W, ((0, 0), (0, EMB_PAD - EMB_DIM)))
    x_pad = jnp.pad(x.astype(jnp.int32), ((0, 0), (0, SEQ_PAD - SEQ)))
    e = _gather_e(emb_pad, x_pad.reshape(-1))
    return _linear(e, w_pad, b)


# speedup vs baseline: 4.9005x; 4.9005x over previous
import jax
import jax.numpy as jnp
from jax import lax
from jax.experimental import pallas as pl
from jax.experimental.pallas import tpu as pltpu
from jax.experimental.pallas import tpu_sc as plsc

NUM_CHARS = 1000
EMB_DIM = 64
EMB_PAD = 128
BATCH = 4096
SEQ = 20
N_EROWS = BATCH * SEQ


_INFO = plsc.get_sparse_core_info()
_NC = _INFO.num_cores
_NS = _INFO.num_subcores
_NW = _NC * _NS
_RW = N_EROWS // _NW
_CHUNK = 128
_NCHUNK = _RW // _CHUNK
_NB = 4
_NITER = _NCHUNK // _NB


def _gather_body(emb_hbm, idx_hbm, e_hbm, idx_v, *scratch):
    bufs = scratch[:_NB]
    gsem = scratch[_NB:2 * _NB]
    ssem = scratch[2 * _NB:3 * _NB]
    wid = lax.axis_index("s") * _NC + lax.axis_index("c")
    base = wid * _RW
    pltpu.sync_copy(idx_hbm.at[pl.ds(base, _RW)], idx_v)

    def g_copy(b, c):
        off = pl.multiple_of(c * _CHUNK, 8)
        return pltpu.make_async_copy(
            emb_hbm.at[idx_v.at[pl.ds(off, _CHUNK)]], bufs[b], gsem[b])

    def s_copy(b, c):
        off = pl.multiple_of(c * _CHUNK, 8)
        return pltpu.make_async_copy(
            bufs[b], e_hbm.at[pl.ds(base + off, _CHUNK)], ssem[b])

    for b in range(_NB):
        g_copy(b, b).start()

    def step(i, fire_next):
        for b in range(_NB):
            c = i * _NB + b
            g_copy(b, c).wait()
            s_copy(b, c).start()
            s_copy(b, c).wait()
            if fire_next:
                g_copy(b, c + _NB).start()
        return 0

    lax.fori_loop(0, _NITER - 1, lambda i, _: step(i, True), 0)
    step(_NITER - 1, False)


def _gather_e(emb_aug, idx_flat):
    mesh = plsc.VectorSubcoreMesh(core_axis_name="c", subcore_axis_name="s")
    return pl.kernel(
        _gather_body,
        out_type=jax.ShapeDtypeStruct((N_EROWS, EMB_PAD), jnp.float32),
        mesh=mesh,
        scratch_types=[
            pltpu.VMEM((_RW,), jnp.int32),
        ] + [pltpu.VMEM((_CHUNK, EMB_PAD), jnp.float32)] * _NB
          + [pltpu.SemaphoreType.DMA] * (2 * _NB),
    )(emb_aug, idx_flat)


_BBLK = 512
_NBBLK = BATCH // _BBLK


def _mm_body(w_ref, e_ref, out_ref):
    out_ref[0] = lax.dot_general(
        w_ref[...], e_ref[...],
        dimension_numbers=(((1,), (1,)), ((), ())),
        preferred_element_type=jnp.float32,
    )


def _linear_t(w_aug, e):
    return pl.pallas_call(
        _mm_body,
        grid=(SEQ, _NBBLK),
        in_specs=[
            pl.BlockSpec((NUM_CHARS, EMB_PAD), lambda s, j: (0, 0)),
            pl.BlockSpec((_BBLK, EMB_PAD),
                         lambda s, j: (s * _NBBLK + j, 0)),
        ],
        out_specs=pl.BlockSpec((1, NUM_CHARS, _BBLK), lambda s, j: (s, 0, j)),
        out_shape=jax.ShapeDtypeStruct((SEQ, NUM_CHARS, BATCH), jnp.float32),
    )(w_aug, e)


def kernel(x, emb_table, W, b):
    emb_aug = jnp.pad(emb_table, ((0, 0), (0, EMB_PAD - EMB_DIM)))
    emb_aug = emb_aug.at[:, EMB_DIM].set(1.0)
    w_aug = jnp.pad(W, ((0, 0), (0, EMB_PAD - EMB_DIM)))
    w_aug = w_aug.at[:, EMB_DIM].set(b)
    idx_flat = x.T.reshape(-1).astype(jnp.int32)
    e = _gather_e(emb_aug, idx_flat)
    out_t = _linear_t(w_aug, e)
    return out_t.transpose(2, 0, 1)
